# Initial kernel scaffold; baseline (speedup 1.0000x reference)
import jax
import jax.numpy as jnp
from jax.experimental import pallas as pl


def kernel(x, llm_feat, W_adapt, b_adapt, W_conv, b_conv, W_ih, W_hh, b_ih, b_hh, gamma, beta, W_proj, b_proj, edges):
    raise NotImplementedError("write your pallas kernel here")


# speedup vs baseline: 4.5357x; 4.5357x over previous
import functools

import jax
import jax.numpy as jnp
from jax import lax
from jax.experimental import pallas as pl
from jax.experimental.pallas import tpu as pltpu
from jax.experimental.pallas import tpu_sc as plsc

N = 10000
E = 320000
R = 2
T = 3
D = 128
RT = R * T

NTILE = 16
NCORE = 2
N_PAD = 10240
RPT = N_PAD // NTILE
EPT = E // NTILE
C = 80
NCHUNK = EPT // C
COMBOS_PER_CORE = RT // NCORE

_mesh = plsc.VectorSubcoreMesh(core_axis_name="c", subcore_axis_name="s")


def _sc_body(g_hbm, src_hbm, dst_hbm, agg_out, deg_out,
             idx_s, idx_d, rows, zbuf, ones_c, zdeg, acc_sh, deg_sh, sem):
    c = lax.axis_index("c")
    s = lax.axis_index("s")

    zv = jnp.zeros((16,), jnp.float32)
    ov = jnp.ones((16,), jnp.float32)

    def zfill(i, _):
        zbuf[i // 8, pl.ds((i % 8) * 16, 16)] = zv
        return 0
    lax.fori_loop(0, C * 8, zfill, 0)

    def ofill(i, _):
        ones_c[pl.ds(i * 16, 16)] = ov
        return 0
    lax.fori_loop(0, C // 16, ofill, 0)

    def zdfill(i, _):
        zdeg[pl.ds(i * 16, 16)] = zv
        return 0
    lax.fori_loop(0, RPT // 16, zdfill, 0)

    for j in range(COMBOS_PER_CORE):
        combo = c * COMBOS_PER_CORE + j
        ebase = combo * E + s * EPT

        for q in range(RPT // C):
            pltpu.sync_copy(zbuf, acc_sh.at[pl.ds(s * RPT + q * C, C)])
        pltpu.sync_copy(zdeg, deg_sh.at[pl.ds(s * RPT, RPT)])
        plsc.subcore_barrier()

        def chunk(i, _):
            base = ebase + i * C
            pltpu.sync_copy(src_hbm.at[pl.ds(base, C)], idx_s)
            pltpu.sync_copy(dst_hbm.at[pl.ds(base, C)], idx_d)
            pltpu.async_copy(g_hbm.at[idx_s], rows, sem).wait()
            pltpu.sync_copy(rows, acc_sh.at[idx_d], add=True)
            pltpu.sync_copy(ones_c, deg_sh.at[idx_d], add=True)
            return 0
        lax.fori_loop(0, NCHUNK, chunk, 0)
        plsc.subcore_barrier()

        pltpu.sync_copy(acc_sh.at[pl.ds(s * RPT, RPT)],
                        agg_out.at[combo, pl.ds(s * RPT, RPT)])
        pltpu.sync_copy(deg_sh.at[pl.ds(s * RPT, RPT)],
                        deg_out.at[combo, pl.ds(s * RPT, RPT)])


_sc_seg = functools.partial(
    pl.kernel,
    out_type=(jax.ShapeDtypeStruct((RT, N_PAD, D), jnp.float32),
              jax.ShapeDtypeStruct((RT, N_PAD), jnp.float32)),
    mesh=_mesh,
    scratch_types=[
        pltpu.VMEM((C,), jnp.int32),
        pltpu.VMEM((C,), jnp.int32),
        pltpu.VMEM((C, D), jnp.float32),
        pltpu.VMEM((C, D), jnp.float32),
        pltpu.VMEM((C,), jnp.float32),
        pltpu.VMEM((RPT,), jnp.float32),
        pltpu.VMEM_SHARED((N_PAD, D), jnp.float32),
        pltpu.VMEM_SHARED((N_PAD,), jnp.float32),
        pltpu.SemaphoreType.DMA,
    ],
)(_sc_body)


def _tc1_body(x_ref, wa_ref, wc_ref, ba_ref, o_ref):
    w2 = jnp.dot(wa_ref[...], wc_ref[...], preferred_element_type=jnp.float32)
    b2 = jnp.dot(ba_ref[...], wc_ref[...], preferred_element_type=jnp.float32)
    o_ref[...] = jnp.dot(x_ref[...], w2, preferred_element_type=jnp.float32) + b2


_TC1_B = 1000

_tc1 = pl.pallas_call(
    _tc1_body,
    grid=(T * N // _TC1_B,),
    in_specs=[pl.BlockSpec((_TC1_B, D), lambda i: (i, 0)),
              pl.BlockSpec((D, D), lambda i: (0, 0)),
              pl.BlockSpec((D, D), lambda i: (0, 0)),
              pl.BlockSpec((1, D), lambda i: (0, 0))],
    out_specs=pl.BlockSpec((_TC1_B, D), lambda i: (i, 0)),
    out_shape=jax.ShapeDtypeStruct((T * N, D), jnp.float32),
)

_NB = N_PAD // RPT


def _feat(agg_ref, deg_ref, cidx, bconv):
    dg = jnp.maximum(deg_ref[cidx], 1.0)
    dg = lax.broadcast_in_dim(dg, (RPT, D), (0,))
    a = agg_ref[cidx] / dg + bconv
    return jnp.where(a > 0, a, jnp.exp(jnp.minimum(a, 0.0)) - 1.0)


def _tc2a_body(agg_ref, deg_ref, wih_ref, p_ref, out_ref):
    pid = pl.program_id(0)
    p = p_ref[...]
    bconv = p[6:7, :]
    valid = (pid * RPT + lax.broadcasted_iota(jnp.int32, (RPT, 1), 0)) < N
    acc = jnp.zeros((8, 128), jnp.float32)
    rows8 = lax.broadcasted_iota(jnp.int32, (8, 128), 0)
    cols8 = lax.broadcasted_iota(jnp.int32, (8, 128), 1)
    for r in range(R):
        h = jnp.full((RPT, 1), 1.0 / R, jnp.float32)
        for t in range(T):
            cidx = r * T + t
            feat = _feat(agg_ref, deg_ref, cidx, bconv)
            gi = [jnp.dot(feat, wih_ref[:, r * 3 + k:r * 3 + k + 1],
                          preferred_element_type=jnp.float32)
                  for k in range(3)]
            gh = [h * p[r, k] + p[4 + r, k] for k in range(3)]
            rg = jax.nn.sigmoid(gi[0] + p[2 + r, 0] + gh[0])
            z = jax.nn.sigmoid(gi[1] + p[2 + r, 1] + gh[1])
            n = jnp.tanh(gi[2] + p[2 + r, 2] + rg * gh[2])
            h = (1.0 - z) * n + z * h
            s_rt = jnp.sum(jnp.where(valid, h, 0.0))
            acc = acc + s_rt * jnp.where((rows8 == r) & (cols8 == t), 1.0, 0.0)

    @pl.when(pid == 0)
    def _():
        out_ref[...] = jnp.zeros((8, 128), jnp.float32)

    out_ref[...] += acc


_tc2a = pl.pallas_call(
    _tc2a_body,
    grid=(_NB,),
    in_specs=[pl.BlockSpec((RT, RPT, D), lambda i: (0, i, 0)),
              pl.BlockSpec((RT, RPT), lambda i: (0, i)),
              pl.BlockSpec((D, R * 3), lambda i: (0, 0)),
              pl.BlockSpec((8, 128), lambda i: (0, 0))],
    out_specs=pl.BlockSpec((8, 128), lambda i: (0, 0)),
    out_shape=jax.ShapeDtypeStruct((8, 128), jnp.float32),
)


def _tc2b_body(agg_ref, deg_ref, p_ref, m_ref, g_ref, b_ref, out_ref):
    p = p_ref[...]
    bconv = p[6:7, :]
    masks = m_ref[...][0:R, 0:T] / float(N)
    mx = jnp.max(masks, axis=0, keepdims=True)
    ex = jnp.exp(masks - mx)
    w = ex / jnp.sum(ex, axis=0, keepdims=True)
    out = jnp.zeros((RPT, D), jnp.float32)
    for t in range(T):
        fused = jnp.zeros((RPT, D), jnp.float32)
        for r in range(R):
            fused = fused + w[r, t] * _feat(agg_ref, deg_ref, r * T + t, bconv)
        mu = jnp.mean(fused, axis=1, keepdims=True)
        cen = fused - mu
        var = jnp.mean(cen * cen, axis=1, keepdims=True)
        ln = cen * lax.rsqrt(var + 1e-5) * g_ref[...] + b_ref[...]
        out = out + p[7, t] * ln
    out_ref[...] = out + p[7, T]


_tc2b = pl.pallas_call(
    _tc2b_body,
    grid=(_NB,),
    in_specs=[pl.BlockSpec((RT, RPT, D), lambda i: (0, i, 0)),
              pl.BlockSpec((RT, RPT), lambda i: (0, i)),
              pl.BlockSpec((8, 128), lambda i: (0, 0)),
              pl.BlockSpec((8, 128), lambda i: (0, 0)),
              pl.BlockSpec((1, D), lambda i: (0, 0)),
              pl.BlockSpec((1, D), lambda i: (0, 0))],
    out_specs=pl.BlockSpec((RPT, D), lambda i: (i, 0)),
    out_shape=jax.ShapeDtypeStruct((N_PAD, D), jnp.float32),
)


def kernel(x, llm_feat, W_adapt, b_adapt, W_conv, b_conv, W_ih, W_hh, b_ih,
           b_hh, gamma, beta, W_proj, b_proj, edges):
    x2 = x.reshape(T * N, D)
    g = _tc1(x2, W_adapt, W_conv, b_adapt.reshape(1, D))

    toff = (jnp.arange(T, dtype=jnp.int32) * N).reshape(1, T, 1)
    src_all = (edges[:, :, 0, :] + toff).reshape(-1)
    dst_all = edges[:, :, 1, :].reshape(-1)
    agg, deg = _sc_seg(g, src_all, dst_all)

    wih_t = W_ih.transpose(2, 0, 1).reshape(D, R * 3)

    P = jnp.zeros((8, 128), jnp.float32)
    P = P.at[0, 0:3].set(W_hh[0, :, 0])
    P = P.at[1, 0:3].set(W_hh[1, :, 0])
    P = P.at[2, 0:3].set(b_ih[0])
    P = P.at[3, 0:3].set(b_ih[1])
    P = P.at[4, 0:3].set(b_hh[0])
    P = P.at[5, 0:3].set(b_hh[1])
    P = P.at[6, 0:D].set(b_conv)
    P = P.at[7, 0:T].set(W_proj)
    P = P.at[7, T].set(b_proj[0])

    masks = _tc2a(agg, deg, wih_t, P)
    out = _tc2b(agg, deg, P, masks, gamma.reshape(1, D), beta.reshape(1, D))
    return out[:N]


# speedup vs baseline: 8.1326x; 1.7930x over previous
import functools

import jax
import jax.numpy as jnp
from jax import lax
from jax.experimental import pallas as pl
from jax.experimental.pallas import tpu as pltpu
from jax.experimental.pallas import tpu_sc as plsc

N = 10000
E = 320000
R = 2
T = 3
D = 128
RT = R * T

NTILE = 16
NCORE = 2
N_PAD = 10240
RPT = N_PAD // NTILE
EPT = E // NTILE
C = 80
NCHUNK = EPT // C
G = 2
NG = NCHUNK // G
COMBOS_PER_CORE = RT // NCORE

_mesh = plsc.VectorSubcoreMesh(core_axis_name="c", subcore_axis_name="s")


def _sc_body(g_hbm, src_hbm, dst_hbm, agg_out, deg_out,
             idx_s, idx_d, rows, ones_c, zdeg, acc_sh, deg_sh, sem):
    c = lax.axis_index("c")
    s = lax.axis_index("s")

    zv = jnp.zeros((16,), jnp.float32)
    ov = jnp.ones((16,), jnp.float32)

    def ofill(i, _):
        ones_c[pl.ds(i * 16, 16)] = ov
        return 0
    lax.fori_loop(0, C // 16, ofill, 0)

    def zdfill(i, _):
        zdeg[pl.ds(i * 16, 16)] = zv
        return 0
    lax.fori_loop(0, RPT // 16, zdfill, 0)

    for j in range(COMBOS_PER_CORE):
        combo = c * COMBOS_PER_CORE + j
        gbase = combo * (E // (G * C)) + s * NG

        def zrows(i, _):
            rows[0, 0, i // 8, pl.ds((i % 8) * 16, 16)] = zv
            return 0
        lax.fori_loop(0, C * 8, zrows, 0)
        for q in range(RPT // C):
            pltpu.sync_copy(rows.at[0, 0], acc_sh.at[pl.ds(s * RPT + q * C, C)])
        pltpu.sync_copy(zdeg, deg_sh.at[pl.ds(s * RPT, RPT)])
        plsc.subcore_barrier()

        def load_group(g, buf):
            pltpu.sync_copy(src_hbm.at[gbase + g], idx_s.at[buf])
            pltpu.sync_copy(dst_hbm.at[gbase + g], idx_d.at[buf])
            for k in range(G):
                pltpu.async_copy(g_hbm.at[idx_s.at[buf, k]],
                                 rows.at[buf, k], sem)

        load_group(0, 0)

        def group(g, _):
            pg = lax.rem(g, 2)
            pn = lax.rem(g + 1, 2)

            @pl.when(g < NG - 1)
            def _():
                load_group(g + 1, pn)

            for k in range(G):
                pltpu.make_async_copy(g_hbm.at[idx_s.at[pg, k]],
                                      rows.at[pg, k], sem).wait()
                pltpu.sync_copy(rows.at[pg, k],
                                acc_sh.at[idx_d.at[pg, k]], add=True)
                pltpu.sync_copy(ones_c, deg_sh.at[idx_d.at[pg, k]], add=True)
            return 0
        lax.fori_loop(0, NG, group, 0)
        plsc.subcore_barrier()

        pltpu.sync_copy(acc_sh.at[pl.ds(s * RPT, RPT)],
                        agg_out.at[combo, pl.ds(s * RPT, RPT)])
        pltpu.sync_copy(deg_sh.at[pl.ds(s * RPT, RPT)],
                        deg_out.at[combo, pl.ds(s * RPT, RPT)])


_sc_seg = functools.partial(
    pl.kernel,
    out_type=(jax.ShapeDtypeStruct((RT, N_PAD, D), jnp.float32),
              jax.ShapeDtypeStruct((RT, N_PAD), jnp.float32)),
    mesh=_mesh,
    scratch_types=[
        pltpu.VMEM((2, G, C), jnp.int32),
        pltpu.VMEM((2, G, C), jnp.int32),
        pltpu.VMEM((2, G, C, D), jnp.float32),
        pltpu.VMEM((C,), jnp.float32),
        pltpu.VMEM((RPT,), jnp.float32),
        pltpu.VMEM_SHARED((N_PAD, D), jnp.float32),
        pltpu.VMEM_SHARED((N_PAD,), jnp.float32),
        pltpu.SemaphoreType.DMA,
    ],
)(_sc_body)


def _tc1_body(x_ref, wa_ref, wc_ref, ba_ref, o_ref):
    w2 = jnp.dot(wa_ref[...], wc_ref[...], preferred_element_type=jnp.float32)
    b2 = jnp.dot(ba_ref[...], wc_ref[...], preferred_element_type=jnp.float32)
    o_ref[...] = jnp.dot(x_ref[...], w2, preferred_element_type=jnp.float32) + b2


_TC1_B = 1000

_tc1 = pl.pallas_call(
    _tc1_body,
    grid=(T * N // _TC1_B,),
    in_specs=[pl.BlockSpec((_TC1_B, D), lambda i: (i, 0)),
              pl.BlockSpec((D, D), lambda i: (0, 0)),
              pl.BlockSpec((D, D), lambda i: (0, 0)),
              pl.BlockSpec((1, D), lambda i: (0, 0))],
    out_specs=pl.BlockSpec((_TC1_B, D), lambda i: (i, 0)),
    out_shape=jax.ShapeDtypeStruct((T * N, D), jnp.float32),
)

_NB = N_PAD // RPT


def _feat(agg_ref, deg_ref, cidx, bconv):
    dg = jnp.maximum(deg_ref[cidx], 1.0)
    dg = lax.broadcast_in_dim(dg, (RPT, D), (0,))
    a = agg_ref[cidx] / dg + bconv
    return jnp.where(a > 0, a, jnp.exp(jnp.minimum(a, 0.0)) - 1.0)


def _tc2a_body(agg_ref, deg_ref, wih_ref, p_ref, out_ref):
    pid = pl.program_id(0)
    p = p_ref[...]
    bconv = p[6:7, :]
    valid = (pid * RPT + lax.broadcasted_iota(jnp.int32, (RPT, 1), 0)) < N
    acc = jnp.zeros((8, 128), jnp.float32)
    rows8 = lax.broadcasted_iota(jnp.int32, (8, 128), 0)
    cols8 = lax.broadcasted_iota(jnp.int32, (8, 128), 1)
    for r in range(R):
        h = jnp.full((RPT, 1), 1.0 / R, jnp.float32)
        for t in range(T):
            cidx = r * T + t
            feat = _feat(agg_ref, deg_ref, cidx, bconv)
            gi = [jnp.dot(feat, wih_ref[:, r * 3 + k:r * 3 + k + 1],
                          preferred_element_type=jnp.float32)
                  for k in range(3)]
            gh = [h * p[r, k] + p[4 + r, k] for k in range(3)]
            rg = jax.nn.sigmoid(gi[0] + p[2 + r, 0] + gh[0])
            z = jax.nn.sigmoid(gi[1] + p[2 + r, 1] + gh[1])
            n = jnp.tanh(gi[2] + p[2 + r, 2] + rg * gh[2])
            h = (1.0 - z) * n + z * h
            s_rt = jnp.sum(jnp.where(valid, h, 0.0))
            acc = acc + s_rt * jnp.where((rows8 == r) & (cols8 == t), 1.0, 0.0)

    @pl.when(pid == 0)
    def _():
        out_ref[...] = jnp.zeros((8, 128), jnp.float32)

    out_ref[...] += acc


_tc2a = pl.pallas_call(
    _tc2a_body,
    grid=(_NB,),
    in_specs=[pl.BlockSpec((RT, RPT, D), lambda i: (0, i, 0)),
              pl.BlockSpec((RT, RPT), lambda i: (0, i)),
              pl.BlockSpec((D, R * 3), lambda i: (0, 0)),
              pl.BlockSpec((8, 128), lambda i: (0, 0))],
    out_specs=pl.BlockSpec((8, 128), lambda i: (0, 0)),
    out_shape=jax.ShapeDtypeStruct((8, 128), jnp.float32),
)


def _tc2b_body(agg_ref, deg_ref, p_ref, m_ref, g_ref, b_ref, out_ref):
    p = p_ref[...]
    bconv = p[6:7, :]
    masks = m_ref[...][0:R, 0:T] / float(N)
    mx = jnp.max(masks, axis=0, keepdims=True)
    ex = jnp.exp(masks - mx)
    w = ex / jnp.sum(ex, axis=0, keepdims=True)
    out = jnp.zeros((RPT, D), jnp.float32)
    for t in range(T):
        fused = jnp.zeros((RPT, D), jnp.float32)
        for r in range(R):
            fused = fused + w[r, t] * _feat(agg_ref, deg_ref, r * T + t, bconv)
        mu = jnp.mean(fused, axis=1, keepdims=True)
        cen = fused - mu
        var = jnp.mean(cen * cen, axis=1, keepdims=True)
        ln = cen * lax.rsqrt(var + 1e-5) * g_ref[...] + b_ref[...]
        out = out + p[7, t] * ln
    out_ref[...] = out + p[7, T]


_tc2b = pl.pallas_call(
    _tc2b_body,
    grid=(_NB,),
    in_specs=[pl.BlockSpec((RT, RPT, D), lambda i: (0, i, 0)),
              pl.BlockSpec((RT, RPT), lambda i: (0, i)),
              pl.BlockSpec((8, 128), lambda i: (0, 0)),
              pl.BlockSpec((8, 128), lambda i: (0, 0)),
              pl.BlockSpec((1, D), lambda i: (0, 0)),
              pl.BlockSpec((1, D), lambda i: (0, 0))],
    out_specs=pl.BlockSpec((RPT, D), lambda i: (i, 0)),
    out_shape=jax.ShapeDtypeStruct((N_PAD, D), jnp.float32),
)


def kernel(x, llm_feat, W_adapt, b_adapt, W_conv, b_conv, W_ih, W_hh, b_ih,
           b_hh, gamma, beta, W_proj, b_proj, edges):
    x2 = x.reshape(T * N, D)
    g = _tc1(x2, W_adapt, W_conv, b_adapt.reshape(1, D))

    toff = (jnp.arange(T, dtype=jnp.int32) * N).reshape(1, T, 1)
    src_all = (edges[:, :, 0, :] + toff).reshape(RT * E // (G * C), G, C)
    dst_all = edges[:, :, 1, :].reshape(RT * E // (G * C), G, C)
    agg, deg = _sc_seg(g, src_all, dst_all)

    wih_t = W_ih.transpose(2, 0, 1).reshape(D, R * 3)

    P = jnp.zeros((8, 128), jnp.float32)
    P = P.at[0, 0:3].set(W_hh[0, :, 0])
    P = P.at[1, 0:3].set(W_hh[1, :, 0])
    P = P.at[2, 0:3].set(b_ih[0])
    P = P.at[3, 0:3].set(b_ih[1])
    P = P.at[4, 0:3].set(b_hh[0])
    P = P.at[5, 0:3].set(b_hh[1])
    P = P.at[6, 0:D].set(b_conv)
    P = P.at[7, 0:T].set(W_proj)
    P = P.at[7, T].set(b_proj[0])

    masks = _tc2a(agg, deg, wih_t, P)
    out = _tc2b(agg, deg, P, masks, gamma.reshape(1, D), beta.reshape(1, D))
    return out[:N]


# speedup vs baseline: 8.1499x; 1.0021x over previous
import functools

import jax
import jax.numpy as jnp
from jax import lax
from jax.experimental import pallas as pl
from jax.experimental.pallas import tpu as pltpu
from jax.experimental.pallas import tpu_sc as plsc

N = 10000
E = 320000
R = 2
T = 3
D = 128
RT = R * T

NTILE = 16
NCORE = 2
N_PAD = 10240
RPT = N_PAD // NTILE
EPT = E // NTILE
C = 80
NCHUNK = EPT // C
G = 2
NG = NCHUNK // G
COMBOS_PER_CORE = RT // NCORE

_mesh = plsc.VectorSubcoreMesh(core_axis_name="c", subcore_axis_name="s")


def _sc_body(g_hbm, src_hbm, dst_hbm, agg_out, deg_out,
             idx_s, idx_d, rows, ones_c, zdeg, acc_sh, deg_sh, sem):
    c = lax.axis_index("c")
    s = lax.axis_index("s")

    zv = jnp.zeros((16,), jnp.float32)
    ov = jnp.ones((16,), jnp.float32)

    def ofill(i, _):
        ones_c[pl.ds(i * 16, 16)] = ov
        return 0
    lax.fori_loop(0, C // 16, ofill, 0)

    def zdfill(i, _):
        zdeg[pl.ds(i * 16, 16)] = zv
        return 0
    lax.fori_loop(0, RPT // 16, zdfill, 0)

    for j in range(COMBOS_PER_CORE):
        combo = c * COMBOS_PER_CORE + j
        gbase = combo * (E // (G * C)) + s * NG

        def zrows(i, _):
            rows[0, 0, i // 8, pl.ds((i % 8) * 16, 16)] = zv
            return 0
        lax.fori_loop(0, C * 8, zrows, 0)
        for q in range(RPT // C):
            pltpu.sync_copy(rows.at[0, 0], acc_sh.at[pl.ds(s * RPT + q * C, C)])
        pltpu.sync_copy(zdeg, deg_sh.at[pl.ds(s * RPT, RPT)])
        plsc.subcore_barrier()

        def load_group(g, buf):
            pltpu.sync_copy(src_hbm.at[gbase + g], idx_s.at[buf])
            pltpu.sync_copy(dst_hbm.at[gbase + g], idx_d.at[buf])
            for k in range(G):
                pltpu.async_copy(g_hbm.at[idx_s.at[buf, k]],
                                 rows.at[buf, k], sem)

        load_group(0, 0)

        def group(g, _):
            pg = lax.rem(g, 2)
            pn = lax.rem(g + 1, 2)

            @pl.when(g < NG - 1)
            def _():
                load_group(g + 1, pn)

            for k in range(G):
                pltpu.make_async_copy(g_hbm.at[idx_s.at[pg, k]],
                                      rows.at[pg, k], sem).wait()
                pltpu.sync_copy(rows.at[pg, k],
                                acc_sh.at[idx_d.at[pg, k]], add=True)
                pltpu.sync_copy(ones_c, deg_sh.at[idx_d.at[pg, k]], add=True)
            return 0
        lax.fori_loop(0, NG, group, 0)
        plsc.subcore_barrier()

        pltpu.sync_copy(acc_sh.at[pl.ds(s * RPT, RPT)],
                        agg_out.at[combo, pl.ds(s * RPT, RPT)])
        pltpu.sync_copy(deg_sh.at[pl.ds(s * RPT, RPT)],
                        deg_out.at[combo, pl.ds(s * RPT, RPT)])


_sc_seg = functools.partial(
    pl.kernel,
    out_type=(jax.ShapeDtypeStruct((RT, N_PAD, D), jnp.float32),
              jax.ShapeDtypeStruct((RT, N_PAD), jnp.float32)),
    mesh=_mesh,
    scratch_types=[
        pltpu.VMEM((2, G, C), jnp.int32),
        pltpu.VMEM((2, G, C), jnp.int32),
        pltpu.VMEM((2, G, C, D), jnp.float32),
        pltpu.VMEM((C,), jnp.float32),
        pltpu.VMEM((RPT,), jnp.float32),
        pltpu.VMEM_SHARED((N_PAD, D), jnp.float32),
        pltpu.VMEM_SHARED((N_PAD,), jnp.float32),
        pltpu.SemaphoreType.DMA,
    ],
)(_sc_body)


_NB = N_PAD // RPT


def _feat(agg_ref, deg_ref, cidx, w2, bac, bconv):
    dg = deg_ref[cidx]
    dgc = lax.broadcast_in_dim(jnp.maximum(dg, 1.0), (RPT, D), (0,))
    ind = lax.broadcast_in_dim(jnp.minimum(dg, 1.0), (RPT, D), (0,))
    a = jnp.dot(agg_ref[cidx], w2, preferred_element_type=jnp.float32) / dgc
    a = a + ind * bac + bconv
    return jnp.where(a > 0, a, jnp.exp(jnp.minimum(a, 0.0)) - 1.0)


def _tc2_body(agg_ref, deg_ref, wa_ref, wc_ref, wih_ref, p_ref, g_ref, b_ref,
              out_ref, msum):
    ph = pl.program_id(0)
    pid = pl.program_id(1)
    p = p_ref[...]
    bconv = p[6:7, :]
    w2 = jnp.dot(wa_ref[...], wc_ref[...], preferred_element_type=jnp.float32)
    bac = jnp.dot(p[8:9, :], wc_ref[...], preferred_element_type=jnp.float32)

    @pl.when(ph == 0)
    def _():
        valid = (pid * RPT
                 + lax.broadcasted_iota(jnp.int32, (RPT, 1), 0)) < N
        acc = jnp.zeros((8, 128), jnp.float32)
        rows8 = lax.broadcasted_iota(jnp.int32, (8, 128), 0)
        cols8 = lax.broadcasted_iota(jnp.int32, (8, 128), 1)
        for r in range(R):
            h = jnp.full((RPT, 1), 1.0 / R, jnp.float32)
            for t in range(T):
                feat = _feat(agg_ref, deg_ref, r * T + t, w2, bac, bconv)
                gi = [jnp.dot(feat, wih_ref[:, r * 3 + k:r * 3 + k + 1],
                              preferred_element_type=jnp.float32)
                      for k in range(3)]
                gh = [h * p[r, k] + p[4 + r, k] for k in range(3)]
                rg = jax.nn.sigmoid(gi[0] + p[2 + r, 0] + gh[0])
                z = jax.nn.sigmoid(gi[1] + p[2 + r, 1] + gh[1])
                n = jnp.tanh(gi[2] + p[2 + r, 2] + rg * gh[2])
                h = (1.0 - z) * n + z * h
                s_rt = jnp.sum(jnp.where(valid, h, 0.0))
                acc = acc + s_rt * jnp.where((rows8 == r) & (cols8 == t),
                                             1.0, 0.0)

        @pl.when(pid == 0)
        def _():
            msum[...] = jnp.zeros((8, 128), jnp.float32)

        msum[...] += acc
        out_ref[...] = jnp.zeros((RPT, D), jnp.float32)

    @pl.when(ph == 1)
    def _():
        masks = msum[...][0:R, 0:T] / float(N)
        mx = jnp.max(masks, axis=0, keepdims=True)
        ex = jnp.exp(masks - mx)
        w = ex / jnp.sum(ex, axis=0, keepdims=True)
        out = jnp.zeros((RPT, D), jnp.float32)
        for t in range(T):
            fused = jnp.zeros((RPT, D), jnp.float32)
            for r in range(R):
                fused = fused + w[r, t] * _feat(agg_ref, deg_ref, r * T + t,
                                                w2, bac, bconv)
            mu = jnp.mean(fused, axis=1, keepdims=True)
            cen = fused - mu
            var = jnp.mean(cen * cen, axis=1, keepdims=True)
            ln = cen * lax.rsqrt(var + 1e-5) * g_ref[...] + b_ref[...]
            out = out + p[7, t] * ln
        out_ref[...] = out + p[7, T]


_tc2 = pl.pallas_call(
    _tc2_body,
    grid=(2, _NB),
    in_specs=[pl.BlockSpec((RT, RPT, D), lambda ph, i: (0, i, 0)),
              pl.BlockSpec((RT, RPT), lambda ph, i: (0, i)),
              pl.BlockSpec((D, D), lambda ph, i: (0, 0)),
              pl.BlockSpec((D, D), lambda ph, i: (0, 0)),
              pl.BlockSpec((D, R * 3), lambda ph, i: (0, 0)),
              pl.BlockSpec((16, 128), lambda ph, i: (0, 0)),
              pl.BlockSpec((1, D), lambda ph, i: (0, 0)),
              pl.BlockSpec((1, D), lambda ph, i: (0, 0))],
    out_specs=pl.BlockSpec((RPT, D), lambda ph, i: (i, 0)),
    out_shape=jax.ShapeDtypeStruct((N_PAD, D), jnp.float32),
    scratch_shapes=[pltpu.VMEM((8, 128), jnp.float32)],
)


def kernel(x, llm_feat, W_adapt, b_adapt, W_conv, b_conv, W_ih, W_hh, b_ih,
           b_hh, gamma, beta, W_proj, b_proj, edges):
    x2 = x.reshape(T * N, D)

    toff = (jnp.arange(T, dtype=jnp.int32) * N).reshape(1, T, 1)
    src_all = (edges[:, :, 0, :] + toff).reshape(RT * E // (G * C), G, C)
    dst_all = edges[:, :, 1, :].reshape(RT * E // (G * C), G, C)
    agg, deg = _sc_seg(x2, src_all, dst_all)

    wih_t = W_ih.transpose(2, 0, 1).reshape(D, R * 3)

    P = jnp.zeros((16, 128), jnp.float32)
    P = P.at[0, 0:3].set(W_hh[0, :, 0])
    P = P.at[1, 0:3].set(W_hh[1, :, 0])
    P = P.at[2, 0:3].set(b_ih[0])
    P = P.at[3, 0:3].set(b_ih[1])
    P = P.at[4, 0:3].set(b_hh[0])
    P = P.at[5, 0:3].set(b_hh[1])
    P = P.at[6, 0:D].set(b_conv)
    P = P.at[7, 0:T].set(W_proj)
    P = P.at[7, T].set(b_proj[0])
    P = P.at[8, 0:D].set(b_adapt)

    out = _tc2(agg, deg, W_adapt, W_conv, wih_t, P,
               gamma.reshape(1, D), beta.reshape(1, D))
    return out[:N]


# speedup vs baseline: 8.1843x; 1.0042x over previous
import functools

import jax
import jax.numpy as jnp
from jax import lax
from jax.experimental import pallas as pl
from jax.experimental.pallas import tpu as pltpu
from jax.experimental.pallas import tpu_sc as plsc

N = 10000
E = 320000
R = 2
T = 3
D = 128
RT = R * T

NTILE = 16
NCORE = 2
N_PAD = 10240
RPT = N_PAD // NTILE
EPT = E // NTILE
C = 80
NCHUNK = EPT // C
G = 2
NG = NCHUNK // G
COMBOS_PER_CORE = RT // NCORE

_mesh = plsc.VectorSubcoreMesh(core_axis_name="c", subcore_axis_name="s")


def _sc_body(g_hbm, src_hbm, dst_hbm, agg_out, deg_out,
             idx_s, idx_d, rows, ones_c, zdeg, acc_sh, deg_sh, sem):
    c = lax.axis_index("c")
    s = lax.axis_index("s")

    zv = jnp.zeros((16,), jnp.float32)
    ov = jnp.ones((16,), jnp.float32)

    def ofill(i, _):
        ones_c[pl.ds(i * 16, 16)] = ov
        return 0
    lax.fori_loop(0, C // 16, ofill, 0)

    def zdfill(i, _):
        zdeg[pl.ds(i * 16, 16)] = zv
        return 0
    lax.fori_loop(0, RPT // 16, zdfill, 0)

    for j in range(COMBOS_PER_CORE):
        combo = c * COMBOS_PER_CORE + j
        gbase = combo * (E // (G * C)) + s * NG

        def zrows(i, _):
            rows[0, 0, i // 8, pl.ds((i % 8) * 16, 16)] = zv
            return 0
        lax.fori_loop(0, C * 8, zrows, 0)
        for q in range(RPT // C):
            pltpu.sync_copy(rows.at[0, 0], acc_sh.at[pl.ds(s * RPT + q * C, C)])
        pltpu.sync_copy(zdeg, deg_sh.at[pl.ds(s * RPT, RPT)])
        plsc.subcore_barrier()

        def load_group(g, buf):
            pltpu.sync_copy(src_hbm.at[gbase + g], idx_s.at[buf])
            pltpu.sync_copy(dst_hbm.at[gbase + g], idx_d.at[buf])
            for k in range(G):
                pltpu.async_copy(g_hbm.at[idx_s.at[buf, k]],
                                 rows.at[buf, k], sem)

        load_group(0, 0)

        def group(g, _):
            pg = lax.rem(g, 2)
            pn = lax.rem(g + 1, 2)

            @pl.when(g < NG - 1)
            def _():
                load_group(g + 1, pn)

            for k in range(G):
                pltpu.make_async_copy(g_hbm.at[idx_s.at[pg, k]],
                                      rows.at[pg, k], sem).wait()
                pltpu.sync_copy(rows.at[pg, k],
                                acc_sh.at[idx_d.at[pg, k]], add=True)
                pltpu.sync_copy(ones_c, deg_sh.at[idx_d.at[pg, k]], add=True)
            return 0
        lax.fori_loop(0, NG, group, 0)
        plsc.subcore_barrier()

        pltpu.sync_copy(acc_sh.at[pl.ds(s * RPT, RPT)],
                        agg_out.at[combo, pl.ds(s * RPT, RPT)])
        pltpu.sync_copy(deg_sh.at[pl.ds(s * RPT, RPT)],
                        deg_out.at[combo, pl.ds(s * RPT, RPT)])


_sc_seg = functools.partial(
    pl.kernel,
    out_type=(jax.ShapeDtypeStruct((RT, N_PAD, D), jnp.float32),
              jax.ShapeDtypeStruct((RT, N_PAD), jnp.float32)),
    mesh=_mesh,
    scratch_types=[
        pltpu.VMEM((2, G, C), jnp.int32),
        pltpu.VMEM((2, G, C), jnp.int32),
        pltpu.VMEM((2, G, C, D), jnp.float32),
        pltpu.VMEM((C,), jnp.float32),
        pltpu.VMEM((RPT,), jnp.float32),
        pltpu.VMEM_SHARED((N_PAD, D), jnp.float32),
        pltpu.VMEM_SHARED((N_PAD,), jnp.float32),
        pltpu.SemaphoreType.DMA,
    ],
)(_sc_body)


_NB = N_PAD // RPT


def _feat(agg_ref, deg_ref, cidx, w2, bac, bconv):
    dg = deg_ref[cidx]
    dgc = lax.broadcast_in_dim(jnp.maximum(dg, 1.0), (RPT, D), (0,))
    ind = lax.broadcast_in_dim(jnp.minimum(dg, 1.0), (RPT, D), (0,))
    a = jnp.dot(agg_ref[cidx], w2, preferred_element_type=jnp.float32) / dgc
    a = a + ind * bac + bconv
    return jnp.where(a > 0, a, jnp.exp(jnp.minimum(a, 0.0)) - 1.0)


def _tca_body(agg_ref, deg_ref, wa_ref, wc_ref, wih_ref, p_ref,
              feats_ref, msum_ref):
    pid = pl.program_id(0)
    p = p_ref[...]
    bconv = p[6:7, :]
    w2 = jnp.dot(wa_ref[...], wc_ref[...], preferred_element_type=jnp.float32)
    bac = jnp.dot(p[8:9, :], wc_ref[...], preferred_element_type=jnp.float32)
    valid = (pid * RPT + lax.broadcasted_iota(jnp.int32, (RPT, 1), 0)) < N
    acc = jnp.zeros((8, 128), jnp.float32)
    rows8 = lax.broadcasted_iota(jnp.int32, (8, 128), 0)
    cols8 = lax.broadcasted_iota(jnp.int32, (8, 128), 1)
    for r in range(R):
        h = jnp.full((RPT, 1), 1.0 / R, jnp.float32)
        for t in range(T):
            cidx = r * T + t
            feat = _feat(agg_ref, deg_ref, cidx, w2, bac, bconv)
            feats_ref[cidx] = feat
            gi3 = jnp.dot(feat, wih_ref[:, r * 3:r * 3 + 3],
                          preferred_element_type=jnp.float32)
            gh = [h * p[r, k] + p[4 + r, k] for k in range(3)]
            rg = jax.nn.sigmoid(gi3[:, 0:1] + p[2 + r, 0] + gh[0])
            z = jax.nn.sigmoid(gi3[:, 1:2] + p[2 + r, 1] + gh[1])
            n = jnp.tanh(gi3[:, 2:3] + p[2 + r, 2] + rg * gh[2])
            h = (1.0 - z) * n + z * h
            s_rt = jnp.sum(jnp.where(valid, h, 0.0))
            acc = acc + s_rt * jnp.where((rows8 == r) & (cols8 == t),
                                         1.0, 0.0)

    @pl.when(pid == 0)
    def _():
        msum_ref[...] = jnp.zeros((8, 128), jnp.float32)

    msum_ref[...] += acc


_tca = pl.pallas_call(
    _tca_body,
    grid=(_NB,),
    in_specs=[pl.BlockSpec((RT, RPT, D), lambda i: (0, i, 0)),
              pl.BlockSpec((RT, RPT), lambda i: (0, i)),
              pl.BlockSpec((D, D), lambda i: (0, 0)),
              pl.BlockSpec((D, D), lambda i: (0, 0)),
              pl.BlockSpec((D, R * 3), lambda i: (0, 0)),
              pl.BlockSpec((16, 128), lambda i: (0, 0))],
    out_specs=[pl.BlockSpec((RT, RPT, D), lambda i: (0, i, 0)),
               pl.BlockSpec((8, 128), lambda i: (0, 0))],
    out_shape=[jax.ShapeDtypeStruct((RT, N_PAD, D), jnp.float32),
               jax.ShapeDtypeStruct((8, 128), jnp.float32)],
)


def _tcb_body(feats_ref, p_ref, m_ref, g_ref, b_ref, out_ref):
    p = p_ref[...]
    masks = m_ref[...][0:R, 0:T] / float(N)
    mx = jnp.max(masks, axis=0, keepdims=True)
    ex = jnp.exp(masks - mx)
    w = ex / jnp.sum(ex, axis=0, keepdims=True)
    out = jnp.zeros((RPT, D), jnp.float32)
    for t in range(T):
        fused = jnp.zeros((RPT, D), jnp.float32)
        for r in range(R):
            fused = fused + w[r, t] * feats_ref[r * T + t]
        mu = jnp.mean(fused, axis=1, keepdims=True)
        cen = fused - mu
        var = jnp.mean(cen * cen, axis=1, keepdims=True)
        ln = cen * lax.rsqrt(var + 1e-5) * g_ref[...] + b_ref[...]
        out = out + p[7, t] * ln
    out_ref[...] = out + p[7, T]


_tcb = pl.pallas_call(
    _tcb_body,
    grid=(_NB,),
    in_specs=[pl.BlockSpec((RT, RPT, D), lambda i: (0, i, 0)),
              pl.BlockSpec((16, 128), lambda i: (0, 0)),
              pl.BlockSpec((8, 128), lambda i: (0, 0)),
              pl.BlockSpec((1, D), lambda i: (0, 0)),
              pl.BlockSpec((1, D), lambda i: (0, 0))],
    out_specs=pl.BlockSpec((RPT, D), lambda i: (i, 0)),
    out_shape=jax.ShapeDtypeStruct((N_PAD, D), jnp.float32),
)


def kernel(x, llm_feat, W_adapt, b_adapt, W_conv, b_conv, W_ih, W_hh, b_ih,
           b_hh, gamma, beta, W_proj, b_proj, edges):
    x2 = x.reshape(T * N, D)

    toff = (jnp.arange(T, dtype=jnp.int32) * N).reshape(1, T, 1)
    src_all = (edges[:, :, 0, :] + toff).reshape(RT * E // (G * C), G, C)
    dst_all = edges[:, :, 1, :].reshape(RT * E // (G * C), G, C)
    agg, deg = _sc_seg(x2, src_all, dst_all)

    wih_t = W_ih.transpose(2, 0, 1).reshape(D, R * 3)

    P = jnp.zeros((16, 128), jnp.float32)
    P = P.at[0, 0:3].set(W_hh[0, :, 0])
    P = P.at[1, 0:3].set(W_hh[1, :, 0])
    P = P.at[2, 0:3].set(b_ih[0])
    P = P.at[3, 0:3].set(b_ih[1])
    P = P.at[4, 0:3].set(b_hh[0])
    P = P.at[5, 0:3].set(b_hh[1])
    P = P.at[6, 0:D].set(b_conv)
    P = P.at[7, 0:T].set(W_proj)
    P = P.at[7, T].set(b_proj[0])
    P = P.at[8, 0:D].set(b_adapt)

    feats, masks = _tca(agg, deg, W_adapt, W_conv, wih_t, P)
    out = _tcb(feats, P, masks, gamma.reshape(1, D), beta.reshape(1, D))
    return out[:N]


# speedup vs baseline: 8.9683x; 1.0958x over previous
import functools

import jax
import jax.numpy as jnp
from jax import lax
from jax.experimental import pallas as pl
from jax.experimental.pallas import tpu as pltpu
from jax.experimental.pallas import tpu_sc as plsc

N = 10000
E = 320000
R = 2
T = 3
D = 128
RT = R * T

NTILE = 16
NCORE = 2
N_PAD = 10240
RPT = N_PAD // NTILE
EPT = E // NTILE
C = 80
NCHUNK = EPT // C
G = 2
NG = NCHUNK // G
COMBOS_PER_CORE = RT // NCORE

_mesh = plsc.VectorSubcoreMesh(core_axis_name="c", subcore_axis_name="s")


def _sc_body(g_hbm, edg_hbm, agg_out, deg_out,
             idx_s, idx_d, rows, ones_c, zdeg, acc_sh, deg_sh, sem):
    c = lax.axis_index("c")
    s = lax.axis_index("s")

    zv = jnp.zeros((16,), jnp.float32)
    ov = jnp.ones((16,), jnp.float32)

    def ofill(i, _):
        ones_c[pl.ds(i * 16, 16)] = ov
        return 0
    lax.fori_loop(0, C // 16, ofill, 0)

    def zdfill(i, _):
        zdeg[pl.ds(i * 16, 16)] = zv
        return 0
    lax.fori_loop(0, RPT // 16, zdfill, 0)

    for j in range(COMBOS_PER_CORE):
        combo = c * COMBOS_PER_CORE + j
        gbase = s * NG
        tof = jnp.full((16,), j * N, jnp.int32)

        def zrows(i, _):
            rows[0, 0, i // 8, pl.ds((i % 8) * 16, 16)] = zv
            return 0
        lax.fori_loop(0, C * 8, zrows, 0)
        for q in range(RPT // C):
            pltpu.sync_copy(rows.at[0, 0], acc_sh.at[pl.ds(s * RPT + q * C, C)])
        pltpu.sync_copy(zdeg, deg_sh.at[pl.ds(s * RPT, RPT)])
        plsc.subcore_barrier()

        def load_group(g, buf):
            pltpu.sync_copy(edg_hbm.at[2 * combo, gbase + g], idx_s.at[buf])
            pltpu.sync_copy(edg_hbm.at[2 * combo + 1, gbase + g],
                            idx_d.at[buf])
            for k in range(G):
                for m in range(C // 16):
                    idx_s[buf, k, pl.ds(m * 16, 16)] += tof
                pltpu.async_copy(g_hbm.at[idx_s.at[buf, k]],
                                 rows.at[buf, k], sem)

        load_group(0, 0)

        def group(g, _):
            pg = lax.rem(g, 2)
            pn = lax.rem(g + 1, 2)

            @pl.when(g < NG - 1)
            def _():
                load_group(g + 1, pn)

            for k in range(G):
                pltpu.make_async_copy(g_hbm.at[idx_s.at[pg, k]],
                                      rows.at[pg, k], sem).wait()
                pltpu.sync_copy(rows.at[pg, k],
                                acc_sh.at[idx_d.at[pg, k]], add=True)
                pltpu.sync_copy(ones_c, deg_sh.at[idx_d.at[pg, k]], add=True)
            return 0
        lax.fori_loop(0, NG, group, 0)
        plsc.subcore_barrier()

        pltpu.sync_copy(acc_sh.at[pl.ds(s * RPT, RPT)],
                        agg_out.at[combo, pl.ds(s * RPT, RPT)])
        pltpu.sync_copy(deg_sh.at[pl.ds(s * RPT, RPT)],
                        deg_out.at[combo, pl.ds(s * RPT, RPT)])


_sc_seg = functools.partial(
    pl.kernel,
    out_type=(jax.ShapeDtypeStruct((RT, N_PAD, D), jnp.float32),
              jax.ShapeDtypeStruct((RT, N_PAD), jnp.float32)),
    mesh=_mesh,
    scratch_types=[
        pltpu.VMEM((2, G, C), jnp.int32),
        pltpu.VMEM((2, G, C), jnp.int32),
        pltpu.VMEM((2, G, C, D), jnp.float32),
        pltpu.VMEM((C,), jnp.float32),
        pltpu.VMEM((RPT,), jnp.float32),
        pltpu.VMEM_SHARED((N_PAD, D), jnp.float32),
        pltpu.VMEM_SHARED((N_PAD,), jnp.float32),
        pltpu.SemaphoreType.DMA,
    ],
)(_sc_body)


_NB = N_PAD // RPT


def _feat(agg_ref, deg_ref, cidx, w2, bac, bconv):
    dg = deg_ref[cidx]
    dgc = lax.broadcast_in_dim(jnp.maximum(dg, 1.0), (RPT, D), (0,))
    ind = lax.broadcast_in_dim(jnp.minimum(dg, 1.0), (RPT, D), (0,))
    a = jnp.dot(agg_ref[cidx], w2, preferred_element_type=jnp.float32) / dgc
    a = a + ind * bac + bconv
    return jnp.where(a > 0, a, jnp.exp(jnp.minimum(a, 0.0)) - 1.0)


def _tca_body(agg_ref, deg_ref, wa_ref, wc_ref, wih_ref, p_ref,
              feats_ref, msum_ref):
    pid = pl.program_id(0)
    p = p_ref[...]
    bconv = p[6:7, :]
    w2 = jnp.dot(wa_ref[...], wc_ref[...], preferred_element_type=jnp.float32)
    bac = jnp.dot(p[8:9, :], wc_ref[...], preferred_element_type=jnp.float32)
    valid = (pid * RPT + lax.broadcasted_iota(jnp.int32, (RPT, 1), 0)) < N
    acc = jnp.zeros((8, 128), jnp.float32)
    rows8 = lax.broadcasted_iota(jnp.int32, (8, 128), 0)
    cols8 = lax.broadcasted_iota(jnp.int32, (8, 128), 1)
    for r in range(R):
        h = jnp.full((RPT, 1), 1.0 / R, jnp.float32)
        for t in range(T):
            cidx = r * T + t
            feat = _feat(agg_ref, deg_ref, cidx, w2, bac, bconv)
            feats_ref[cidx] = feat
            gi3 = jnp.dot(feat, wih_ref[:, r * 3:r * 3 + 3],
                          preferred_element_type=jnp.float32)
            gh = [h * p[r, k] + p[4 + r, k] for k in range(3)]
            rg = jax.nn.sigmoid(gi3[:, 0:1] + p[2 + r, 0] + gh[0])
            z = jax.nn.sigmoid(gi3[:, 1:2] + p[2 + r, 1] + gh[1])
            n = jnp.tanh(gi3[:, 2:3] + p[2 + r, 2] + rg * gh[2])
            h = (1.0 - z) * n + z * h
            s_rt = jnp.sum(jnp.where(valid, h, 0.0))
            acc = acc + s_rt * jnp.where((rows8 == r) & (cols8 == t),
                                         1.0, 0.0)

    @pl.when(pid == 0)
    def _():
        msum_ref[...] = jnp.zeros((8, 128), jnp.float32)

    msum_ref[...] += acc


_tca = pl.pallas_call(
    _tca_body,
    grid=(_NB,),
    in_specs=[pl.BlockSpec((RT, RPT, D), lambda i: (0, i, 0)),
              pl.BlockSpec((RT, RPT), lambda i: (0, i)),
              pl.BlockSpec((D, D), lambda i: (0, 0)),
              pl.BlockSpec((D, D), lambda i: (0, 0)),
              pl.BlockSpec((D, R * 3), lambda i: (0, 0)),
              pl.BlockSpec((16, 128), lambda i: (0, 0))],
    out_specs=[pl.BlockSpec((RT, RPT, D), lambda i: (0, i, 0)),
               pl.BlockSpec((8, 128), lambda i: (0, 0))],
    out_shape=[jax.ShapeDtypeStruct((RT, N_PAD, D), jnp.float32),
               jax.ShapeDtypeStruct((8, 128), jnp.float32)],
)


def _tcb_body(feats_ref, p_ref, m_ref, g_ref, b_ref, out_ref):
    p = p_ref[...]
    masks = m_ref[...][0:R, 0:T] / float(N)
    mx = jnp.max(masks, axis=0, keepdims=True)
    ex = jnp.exp(masks - mx)
    w = ex / jnp.sum(ex, axis=0, keepdims=True)
    out = jnp.zeros((RPT, D), jnp.float32)
    for t in range(T):
        fused = jnp.zeros((RPT, D), jnp.float32)
        for r in range(R):
            fused = fused + w[r, t] * feats_ref[r * T + t]
        mu = jnp.mean(fused, axis=1, keepdims=True)
        cen = fused - mu
        var = jnp.mean(cen * cen, axis=1, keepdims=True)
        ln = cen * lax.rsqrt(var + 1e-5) * g_ref[...] + b_ref[...]
        out = out + p[7, t] * ln
    out_ref[...] = out + p[7, T]


_tcb = pl.pallas_call(
    _tcb_body,
    grid=(_NB,),
    in_specs=[pl.BlockSpec((RT, RPT, D), lambda i: (0, i, 0)),
              pl.BlockSpec((16, 128), lambda i: (0, 0)),
              pl.BlockSpec((8, 128), lambda i: (0, 0)),
              pl.BlockSpec((1, D), lambda i: (0, 0)),
              pl.BlockSpec((1, D), lambda i: (0, 0))],
    out_specs=pl.BlockSpec((RPT, D), lambda i: (i, 0)),
    out_shape=jax.ShapeDtypeStruct((N_PAD, D), jnp.float32),
)


def kernel(x, llm_feat, W_adapt, b_adapt, W_conv, b_conv, W_ih, W_hh, b_ih,
           b_hh, gamma, beta, W_proj, b_proj, edges):
    x2 = x.reshape(T * N, D)

    edg = edges.reshape(RT * 2, E // (G * C), G, C)
    agg, deg = _sc_seg(x2, edg)

    wih_t = W_ih.transpose(2, 0, 1).reshape(D, R * 3)

    P = jnp.zeros((16, 128), jnp.float32)
    P = P.at[0, 0:3].set(W_hh[0, :, 0])
    P = P.at[1, 0:3].set(W_hh[1, :, 0])
    P = P.at[2, 0:3].set(b_ih[0])
    P = P.at[3, 0:3].set(b_ih[1])
    P = P.at[4, 0:3].set(b_hh[0])
    P = P.at[5, 0:3].set(b_hh[1])
    P = P.at[6, 0:D].set(b_conv)
    P = P.at[7, 0:T].set(W_proj)
    P = P.at[7, T].set(b_proj[0])
    P = P.at[8, 0:D].set(b_adapt)

    feats, masks = _tca(agg, deg, W_adapt, W_conv, wih_t, P)
    out = _tcb(feats, P, masks, gamma.reshape(1, D), beta.reshape(1, D))
    return out[:N]


# speedup vs baseline: 9.2420x; 1.0305x over previous
import functools

import jax
import jax.numpy as jnp
from jax import lax
from jax.experimental import pallas as pl
from jax.experimental.pallas import tpu as pltpu
from jax.experimental.pallas import tpu_sc as plsc

N = 10000
E = 320000
R = 2
T = 3
D = 128
RT = R * T

NTILE = 16
NCORE = 2
N_PAD = 10240
RPT = N_PAD // NTILE
EPT = E // NTILE
C = 80
NCHUNK = EPT // C
G = 2
NG = NCHUNK // G
COMBOS_PER_CORE = RT // NCORE

_mesh = plsc.VectorSubcoreMesh(core_axis_name="c", subcore_axis_name="s")


def _make_sc_body(t):
    def _sc_body(g_hbm, edg_hbm, agg_out, deg_out,
                 idx_s, idx_d, rows, ones_c, zdeg, acc_sh, deg_sh, sem):
        c = lax.axis_index("c")
        s = lax.axis_index("s")

        zv = jnp.zeros((16,), jnp.float32)
        ov = jnp.ones((16,), jnp.float32)

        def ofill(i, _):
            ones_c[pl.ds(i * 16, 16)] = ov
            return 0
        lax.fori_loop(0, C // 16, ofill, 0)

        def zdfill(i, _):
            zdeg[pl.ds(i * 16, 16)] = zv
            return 0
        lax.fori_loop(0, RPT // 16, zdfill, 0)

        gbase = s * NG
        tof = jnp.full((16,), t * N, jnp.int32)

        def zrows(i, _):
            rows[0, 0, i // 8, pl.ds((i % 8) * 16, 16)] = zv
            return 0
        lax.fori_loop(0, C * 8, zrows, 0)
        for q in range(RPT // C):
            pltpu.sync_copy(rows.at[0, 0],
                            acc_sh.at[pl.ds(s * RPT + q * C, C)])
        pltpu.sync_copy(zdeg, deg_sh.at[pl.ds(s * RPT, RPT)])
        plsc.subcore_barrier()

        def load_group(g, buf):
            pltpu.sync_copy(edg_hbm.at[2 * c, gbase + g], idx_s.at[buf])
            pltpu.sync_copy(edg_hbm.at[2 * c + 1, gbase + g], idx_d.at[buf])
            for k in range(G):
                for m in range(C // 16):
                    idx_s[buf, k, pl.ds(m * 16, 16)] += tof
                pltpu.async_copy(g_hbm.at[idx_s.at[buf, k]],
                                 rows.at[buf, k], sem)

        load_group(0, 0)

        def group(g, _):
            pg = lax.rem(g, 2)
            pn = lax.rem(g + 1, 2)

            @pl.when(g < NG - 1)
            def _():
                load_group(g + 1, pn)

            for k in range(G):
                pltpu.make_async_copy(g_hbm.at[idx_s.at[pg, k]],
                                      rows.at[pg, k], sem).wait()
                pltpu.sync_copy(rows.at[pg, k],
                                acc_sh.at[idx_d.at[pg, k]], add=True)
                pltpu.sync_copy(ones_c, deg_sh.at[idx_d.at[pg, k]], add=True)
            return 0
        lax.fori_loop(0, NG, group, 0)
        plsc.subcore_barrier()

        pltpu.sync_copy(acc_sh.at[pl.ds(s * RPT, RPT)],
                        agg_out.at[c, pl.ds(s * RPT, RPT)])
        pltpu.sync_copy(deg_sh.at[pl.ds(s * RPT, RPT)],
                        deg_out.at[c, pl.ds(s * RPT, RPT)])
    return _sc_body


def _make_sc(t):
    return functools.partial(
        pl.kernel,
        out_type=(jax.ShapeDtypeStruct((R, N_PAD, D), jnp.float32),
                  jax.ShapeDtypeStruct((R, N_PAD), jnp.float32)),
        mesh=_mesh,
        scratch_types=[
            pltpu.VMEM((2, G, C), jnp.int32),
            pltpu.VMEM((2, G, C), jnp.int32),
            pltpu.VMEM((2, G, C, D), jnp.float32),
            pltpu.VMEM((C,), jnp.float32),
            pltpu.VMEM((RPT,), jnp.float32),
            pltpu.VMEM_SHARED((N_PAD, D), jnp.float32),
            pltpu.VMEM_SHARED((N_PAD,), jnp.float32),
            pltpu.SemaphoreType.DMA,
        ],
    )(_make_sc_body(t))


_sc_seg_t = [_make_sc(t) for t in range(T)]


_NB = N_PAD // RPT


def _feat(agg_ref, deg_ref, r, w2, bac, bconv):
    dg = deg_ref[r]
    dgc = lax.broadcast_in_dim(jnp.maximum(dg, 1.0), (RPT, D), (0,))
    ind = lax.broadcast_in_dim(jnp.minimum(dg, 1.0), (RPT, D), (0,))
    a = jnp.dot(agg_ref[r], w2, preferred_element_type=jnp.float32) / dgc
    a = a + ind * bac + bconv
    return jnp.where(a > 0, a, jnp.exp(jnp.minimum(a, 0.0)) - 1.0)


def _tca_body(agg0, agg1, agg2, deg0, deg1, deg2, wa_ref, wc_ref, wih_ref,
              p_ref, feats_ref, msum_ref):
    aggs = [agg0, agg1, agg2]
    degs = [deg0, deg1, deg2]
    pid = pl.program_id(0)
    p = p_ref[...]
    bconv = p[6:7, :]
    w2 = jnp.dot(wa_ref[...], wc_ref[...], preferred_element_type=jnp.float32)
    bac = jnp.dot(p[8:9, :], wc_ref[...], preferred_element_type=jnp.float32)
    valid = (pid * RPT + lax.broadcasted_iota(jnp.int32, (RPT, 1), 0)) < N
    acc = jnp.zeros((8, 128), jnp.float32)
    rows8 = lax.broadcasted_iota(jnp.int32, (8, 128), 0)
    cols8 = lax.broadcasted_iota(jnp.int32, (8, 128), 1)
    for r in range(R):
        h = jnp.full((RPT, 1), 1.0 / R, jnp.float32)
        for t in range(T):
            feat = _feat(aggs[t], degs[t], r, w2, bac, bconv)
            feats_ref[r * T + t] = feat
            gi3 = jnp.dot(feat, wih_ref[:, r * 3:r * 3 + 3],
                          preferred_element_type=jnp.float32)
            gh = [h * p[r, k] + p[4 + r, k] for k in range(3)]
            rg = jax.nn.sigmoid(gi3[:, 0:1] + p[2 + r, 0] + gh[0])
            z = jax.nn.sigmoid(gi3[:, 1:2] + p[2 + r, 1] + gh[1])
            n = jnp.tanh(gi3[:, 2:3] + p[2 + r, 2] + rg * gh[2])
            h = (1.0 - z) * n + z * h
            s_rt = jnp.sum(jnp.where(valid, h, 0.0))
            acc = acc + s_rt * jnp.where((rows8 == r) & (cols8 == t),
                                         1.0, 0.0)

    @pl.when(pid == 0)
    def _():
        msum_ref[...] = jnp.zeros((8, 128), jnp.float32)

    msum_ref[...] += acc


_tca = pl.pallas_call(
    _tca_body,
    grid=(_NB,),
    in_specs=[pl.BlockSpec((R, RPT, D), lambda i: (0, i, 0)),
              pl.BlockSpec((R, RPT, D), lambda i: (0, i, 0)),
              pl.BlockSpec((R, RPT, D), lambda i: (0, i, 0)),
              pl.BlockSpec((R, RPT), lambda i: (0, i)),
              pl.BlockSpec((R, RPT), lambda i: (0, i)),
              pl.BlockSpec((R, RPT), lambda i: (0, i)),
              pl.BlockSpec((D, D), lambda i: (0, 0)),
              pl.BlockSpec((D, D), lambda i: (0, 0)),
              pl.BlockSpec((D, R * 3), lambda i: (0, 0)),
              pl.BlockSpec((16, 128), lambda i: (0, 0))],
    out_specs=[pl.BlockSpec((RT, RPT, D), lambda i: (0, i, 0)),
               pl.BlockSpec((8, 128), lambda i: (0, 0))],
    out_shape=[jax.ShapeDtypeStruct((RT, N_PAD, D), jnp.float32),
               jax.ShapeDtypeStruct((8, 128), jnp.float32)],
)


def _tcb_body(feats_ref, p_ref, m_ref, g_ref, b_ref, out_ref):
    p = p_ref[...]
    masks = m_ref[...][0:R, 0:T] / float(N)
    mx = jnp.max(masks, axis=0, keepdims=True)
    ex = jnp.exp(masks - mx)
    w = ex / jnp.sum(ex, axis=0, keepdims=True)
    out = jnp.zeros((RPT, D), jnp.float32)
    for t in range(T):
        fused = jnp.zeros((RPT, D), jnp.float32)
        for r in range(R):
            fused = fused + w[r, t] * feats_ref[r * T + t]
        mu = jnp.mean(fused, axis=1, keepdims=True)
        cen = fused - mu
        var = jnp.mean(cen * cen, axis=1, keepdims=True)
        ln = cen * lax.rsqrt(var + 1e-5) * g_ref[...] + b_ref[...]
        out = out + p[7, t] * ln
    out_ref[...] = out + p[7, T]


_tcb = pl.pallas_call(
    _tcb_body,
    grid=(_NB,),
    in_specs=[pl.BlockSpec((RT, RPT, D), lambda i: (0, i, 0)),
              pl.BlockSpec((16, 128), lambda i: (0, 0)),
              pl.BlockSpec((8, 128), lambda i: (0, 0)),
              pl.BlockSpec((1, D), lambda i: (0, 0)),
              pl.BlockSpec((1, D), lambda i: (0, 0))],
    out_specs=pl.BlockSpec((RPT, D), lambda i: (i, 0)),
    out_shape=jax.ShapeDtypeStruct((N_PAD, D), jnp.float32),
)


def kernel(x, llm_feat, W_adapt, b_adapt, W_conv, b_conv, W_ih, W_hh, b_ih,
           b_hh, gamma, beta, W_proj, b_proj, edges):
    x2 = x.reshape(T * N, D)

    aggs, degs = [], []
    for t in range(T):
        edg_t = edges[:, t].reshape(R * 2, E // (G * C), G, C)
        agg_t, deg_t = _sc_seg_t[t](x2, edg_t)
        aggs.append(agg_t)
        degs.append(deg_t)

    wih_t = W_ih.transpose(2, 0, 1).reshape(D, R * 3)

    P = jnp.zeros((16, 128), jnp.float32)
    P = P.at[0, 0:3].set(W_hh[0, :, 0])
    P = P.at[1, 0:3].set(W_hh[1, :, 0])
    P = P.at[2, 0:3].set(b_ih[0])
    P = P.at[3, 0:3].set(b_ih[1])
    P = P.at[4, 0:3].set(b_hh[0])
    P = P.at[5, 0:3].set(b_hh[1])
    P = P.at[6, 0:D].set(b_conv)
    P = P.at[7, 0:T].set(W_proj)
    P = P.at[7, T].set(b_proj[0])
    P = P.at[8, 0:D].set(b_adapt)

    feats, masks = _tca(aggs[0], aggs[1], aggs[2], degs[0], degs[1], degs[2],
                        W_adapt, W_conv, wih_t, P)
    out = _tcb(feats, P, masks, gamma.reshape(1, D), beta.reshape(1, D))
    return out[:N]


# speedup vs baseline: 9.6325x; 1.0422x over previous
import functools

import jax
import jax.numpy as jnp
from jax import lax
from jax.experimental import pallas as pl
from jax.experimental.pallas import tpu as pltpu
from jax.experimental.pallas import tpu_sc as plsc

N = 10000
E = 320000
R = 2
T = 3
D = 128
RT = R * T

NTILE = 16
NCORE = 2
N_PAD = 10240
RPT = N_PAD // NTILE
EPT = E // NTILE
C = 80
NCHUNK = EPT // C
G = 2
NG = NCHUNK // G
COMBOS_PER_CORE = RT // NCORE

_mesh = plsc.VectorSubcoreMesh(core_axis_name="c", subcore_axis_name="s")


def _make_sc_body(t):
    def _sc_body(g_hbm, edg_hbm, agg_out, deg_out,
                 idx_s, idx_d, rows, ones_c, zdeg, acc_sh, deg_sh, sem):
        c = lax.axis_index("c")
        s = lax.axis_index("s")

        zv = jnp.zeros((16,), jnp.float32)
        ov = jnp.ones((16,), jnp.float32)

        def ofill(i, _):
            ones_c[pl.ds(i * 16, 16)] = ov
            return 0
        lax.fori_loop(0, C // 16, ofill, 0)

        def zdfill(i, _):
            zdeg[pl.ds(i * 16, 16)] = zv
            return 0
        lax.fori_loop(0, RPT // 16, zdfill, 0)

        gbase = s * NG
        tof = jnp.full((16,), t * N, jnp.int32)

        def zrows(i, _):
            rows[0, 0, i // 8, pl.ds((i % 8) * 16, 16)] = zv
            return 0
        lax.fori_loop(0, C * 8, zrows, 0)
        for q in range(RPT // C):
            pltpu.sync_copy(rows.at[0, 0],
                            acc_sh.at[pl.ds(s * RPT + q * C, C)])
        pltpu.sync_copy(zdeg, deg_sh.at[pl.ds(s * RPT, RPT)])
        plsc.subcore_barrier()

        def load_group(g, buf):
            pltpu.sync_copy(edg_hbm.at[2 * c, gbase + g], idx_s.at[buf])
            pltpu.sync_copy(edg_hbm.at[2 * c + 1, gbase + g], idx_d.at[buf])
            for k in range(G):
                for m in range(C // 16):
                    idx_s[buf, k, pl.ds(m * 16, 16)] += tof
                pltpu.async_copy(g_hbm.at[idx_s.at[buf, k]],
                                 rows.at[buf, k], sem)

        load_group(0, 0)

        def group(g, _):
            pg = lax.rem(g, 2)
            pn = lax.rem(g + 1, 2)

            @pl.when(g < NG - 1)
            def _():
                load_group(g + 1, pn)

            for k in range(G):
                pltpu.make_async_copy(g_hbm.at[idx_s.at[pg, k]],
                                      rows.at[pg, k], sem).wait()
                pltpu.sync_copy(rows.at[pg, k],
                                acc_sh.at[idx_d.at[pg, k]], add=True)
                pltpu.sync_copy(ones_c, deg_sh.at[idx_d.at[pg, k]], add=True)
            return 0
        lax.fori_loop(0, NG, group, 0)
        plsc.subcore_barrier()

        pltpu.sync_copy(acc_sh.at[pl.ds(s * RPT, RPT)],
                        agg_out.at[c, pl.ds(s * RPT, RPT)])
        pltpu.sync_copy(deg_sh.at[pl.ds(s * RPT, RPT)],
                        deg_out.at[c, pl.ds(s * RPT, RPT)])
    return _sc_body


def _make_sc(t):
    return functools.partial(
        pl.kernel,
        out_type=(jax.ShapeDtypeStruct((R, N_PAD, D), jnp.float32),
                  jax.ShapeDtypeStruct((R, N_PAD), jnp.float32)),
        mesh=_mesh,
        scratch_types=[
            pltpu.VMEM((2, G, C), jnp.int32),
            pltpu.VMEM((2, G, C), jnp.int32),
            pltpu.VMEM((2, G, C, D), jnp.float32),
            pltpu.VMEM((C,), jnp.float32),
            pltpu.VMEM((RPT,), jnp.float32),
            pltpu.VMEM_SHARED((N_PAD, D), jnp.float32),
            pltpu.VMEM_SHARED((N_PAD,), jnp.float32),
            pltpu.SemaphoreType.DMA,
        ],
    )(_make_sc_body(t))


_sc_seg_t = [_make_sc(t) for t in range(T)]


_NB = N_PAD // RPT


def _feat(agg_ref, deg_ref, r, w2, bac, bconv):
    dg = deg_ref[r]
    dgc = lax.broadcast_in_dim(jnp.maximum(dg, 1.0), (RPT, D), (0,))
    ind = lax.broadcast_in_dim(jnp.minimum(dg, 1.0), (RPT, D), (0,))
    a = jnp.dot(agg_ref[r], w2, preferred_element_type=jnp.float32) / dgc
    a = a + ind * bac + bconv
    return jnp.where(a > 0, a, jnp.exp(jnp.minimum(a, 0.0)) - 1.0)


def _make_tca_body(t):
    def body(agg, deg, wa_ref, wc_ref, wih_ref, p_ref, *rest):
        if t == 0:
            feats_ref, h_out, msum_out = rest
            h_in = msum_in = None
        else:
            h_in, msum_in, feats_ref, h_out, msum_out = rest
        pid = pl.program_id(0)
        p = p_ref[...]
        bconv = p[6:7, :]
        w2 = jnp.dot(wa_ref[...], wc_ref[...],
                     preferred_element_type=jnp.float32)
        bac = jnp.dot(p[8:9, :], wc_ref[...],
                      preferred_element_type=jnp.float32)
        valid = (pid * RPT
                 + lax.broadcasted_iota(jnp.int32, (RPT, 1), 0)) < N
        acc = jnp.zeros((8, 128), jnp.float32)
        rows8 = lax.broadcasted_iota(jnp.int32, (8, 128), 0)
        cols8 = lax.broadcasted_iota(jnp.int32, (8, 128), 1)
        h_cols = []
        for r in range(R):
            if t == 0:
                h = jnp.full((RPT, 1), 1.0 / R, jnp.float32)
            else:
                h = h_in[...][:, r:r + 1]
            feat = _feat(agg, deg, r, w2, bac, bconv)
            feats_ref[r] = feat
            gi3 = jnp.dot(feat, wih_ref[:, r * 3:r * 3 + 3],
                          preferred_element_type=jnp.float32)
            gh = [h * p[r, k] + p[4 + r, k] for k in range(3)]
            rg = jax.nn.sigmoid(gi3[:, 0:1] + p[2 + r, 0] + gh[0])
            z = jax.nn.sigmoid(gi3[:, 1:2] + p[2 + r, 1] + gh[1])
            n = jnp.tanh(gi3[:, 2:3] + p[2 + r, 2] + rg * gh[2])
            h = (1.0 - z) * n + z * h
            h_cols.append(h)
            s_rt = jnp.sum(jnp.where(valid, h, 0.0))
            acc = acc + s_rt * jnp.where((rows8 == r) & (cols8 == t),
                                         1.0, 0.0)
        h_out[...] = jnp.concatenate(h_cols, axis=1)

        @pl.when(pid == 0)
        def _():
            if t == 0:
                msum_out[...] = jnp.zeros((8, 128), jnp.float32)
            else:
                msum_out[...] = msum_in[...]

        msum_out[...] += acc
    return body


def _make_tca(t):
    in_specs = [pl.BlockSpec((R, RPT, D), lambda i: (0, i, 0)),
                pl.BlockSpec((R, RPT), lambda i: (0, i)),
                pl.BlockSpec((D, D), lambda i: (0, 0)),
                pl.BlockSpec((D, D), lambda i: (0, 0)),
                pl.BlockSpec((D, R * 3), lambda i: (0, 0)),
                pl.BlockSpec((16, 128), lambda i: (0, 0))]
    if t > 0:
        in_specs += [pl.BlockSpec((RPT, R), lambda i: (i, 0)),
                     pl.BlockSpec((8, 128), lambda i: (0, 0))]
    return pl.pallas_call(
        _make_tca_body(t),
        grid=(_NB,),
        in_specs=in_specs,
        out_specs=[pl.BlockSpec((R, RPT, D), lambda i: (0, i, 0)),
                   pl.BlockSpec((RPT, R), lambda i: (i, 0)),
                   pl.BlockSpec((8, 128), lambda i: (0, 0))],
        out_shape=[jax.ShapeDtypeStruct((R, N_PAD, D), jnp.float32),
                   jax.ShapeDtypeStruct((N_PAD, R), jnp.float32),
                   jax.ShapeDtypeStruct((8, 128), jnp.float32)],
    )


_tca_t = [_make_tca(t) for t in range(T)]


def _tcb_body(f0, f1, f2, p_ref, m_ref, g_ref, b_ref, out_ref):
    feats = [f0, f1, f2]
    p = p_ref[...]
    masks = m_ref[...][0:R, 0:T] / float(N)
    mx = jnp.max(masks, axis=0, keepdims=True)
    ex = jnp.exp(masks - mx)
    w = ex / jnp.sum(ex, axis=0, keepdims=True)
    out = jnp.zeros((RPT, D), jnp.float32)
    for t in range(T):
        fused = jnp.zeros((RPT, D), jnp.float32)
        for r in range(R):
            fused = fused + w[r, t] * feats[t][r]
        mu = jnp.mean(fused, axis=1, keepdims=True)
        cen = fused - mu
        var = jnp.mean(cen * cen, axis=1, keepdims=True)
        ln = cen * lax.rsqrt(var + 1e-5) * g_ref[...] + b_ref[...]
        out = out + p[7, t] * ln
    out_ref[...] = out + p[7, T]


_tcb = pl.pallas_call(
    _tcb_body,
    grid=(_NB,),
    in_specs=[pl.BlockSpec((R, RPT, D), lambda i: (0, i, 0)),
              pl.BlockSpec((R, RPT, D), lambda i: (0, i, 0)),
              pl.BlockSpec((R, RPT, D), lambda i: (0, i, 0)),
              pl.BlockSpec((16, 128), lambda i: (0, 0)),
              pl.BlockSpec((8, 128), lambda i: (0, 0)),
              pl.BlockSpec((1, D), lambda i: (0, 0)),
              pl.BlockSpec((1, D), lambda i: (0, 0))],
    out_specs=pl.BlockSpec((RPT, D), lambda i: (i, 0)),
    out_shape=jax.ShapeDtypeStruct((N_PAD, D), jnp.float32),
)


def kernel(x, llm_feat, W_adapt, b_adapt, W_conv, b_conv, W_ih, W_hh, b_ih,
           b_hh, gamma, beta, W_proj, b_proj, edges):
    x2 = x.reshape(T * N, D)

    aggs, degs = [], []
    for t in range(T):
        edg_t = edges[:, t].reshape(R * 2, E // (G * C), G, C)
        agg_t, deg_t = _sc_seg_t[t](x2, edg_t)
        aggs.append(agg_t)
        degs.append(deg_t)

    wih_t = W_ih.transpose(2, 0, 1).reshape(D, R * 3)

    P = jnp.zeros((16, 128), jnp.float32)
    P = P.at[0, 0:3].set(W_hh[0, :, 0])
    P = P.at[1, 0:3].set(W_hh[1, :, 0])
    P = P.at[2, 0:3].set(b_ih[0])
    P = P.at[3, 0:3].set(b_ih[1])
    P = P.at[4, 0:3].set(b_hh[0])
    P = P.at[5, 0:3].set(b_hh[1])
    P = P.at[6, 0:D].set(b_conv)
    P = P.at[7, 0:T].set(W_proj)
    P = P.at[7, T].set(b_proj[0])
    P = P.at[8, 0:D].set(b_adapt)

    feats0, h, msum = _tca_t[0](aggs[0], degs[0], W_adapt, W_conv, wih_t, P)
    feats1, h, msum = _tca_t[1](aggs[1], degs[1], W_adapt, W_conv, wih_t, P,
                                h, msum)
    feats2, h, msum = _tca_t[2](aggs[2], degs[2], W_adapt, W_conv, wih_t, P,
                                h, msum)
    out = _tcb(feats0, feats1, feats2, P, msum,
               gamma.reshape(1, D), beta.reshape(1, D))
    return out[:N]
